# baseline (speedup 1.0000x reference)
import functools

import jax
import jax.numpy as jnp
from jax.experimental import pallas as pl
from jax.experimental.pallas import tpu as pltpu


def _mlp_block(x_ref, w1_ref, b1_ref, w2_ref, b2_ref, w3_ref, b3_ref, z_ref):
    h = jax.nn.relu(jnp.dot(x_ref[...], w1_ref[...], preferred_element_type=jnp.float32) + b1_ref[...])
    h = jax.nn.relu(jnp.dot(h, w2_ref[...], preferred_element_type=jnp.float32) + b2_ref[...])
    z_ref[...] = jnp.dot(h, w3_ref[...], preferred_element_type=jnp.float32) + b3_ref[...]


def _mlp(x, W1, b1, W2, b2, W3p, b3p):
    n = x.shape[0]
    BLK = 1000
    grid = (n // BLK,)
    return pl.pallas_call(
        _mlp_block,
        grid=grid,
        in_specs=[
            pl.BlockSpec((BLK, 128), lambda i: (i, 0)),
            pl.BlockSpec((128, 256), lambda i: (0, 0)),
            pl.BlockSpec((256,), lambda i: (0,)),
            pl.BlockSpec((256, 256), lambda i: (0, 0)),
            pl.BlockSpec((256,), lambda i: (0,)),
            pl.BlockSpec((256, 128), lambda i: (0, 0)),
            pl.BlockSpec((128,), lambda i: (0,)),
        ],
        out_specs=pl.BlockSpec((BLK, 128), lambda i: (i, 0)),
        out_shape=jax.ShapeDtypeStruct((n, 128), jnp.float32),
    )(x, W1, b1, W2, b2, W3p, b3p)


def _gat(x, src, dst, W, a_src, a_dst, b, heads, out_ch, n):
    h = (x @ W).reshape(n, heads, out_ch)
    al_s = jnp.sum(h * a_src[None], axis=-1)
    al_d = jnp.sum(h * a_dst[None], axis=-1)
    e = jax.nn.leaky_relu(al_s[src] + al_d[dst], negative_slope=0.2)
    m = jax.lax.stop_gradient(jax.ops.segment_max(e, dst, num_segments=n))
    ex = jnp.exp(e - m[dst])
    s = jax.ops.segment_sum(ex, dst, num_segments=n)
    alpha = ex / (s[dst] + 1e-16)
    msg = h[src] * alpha[:, :, None]
    agg = jax.ops.segment_sum(msg, dst, num_segments=n)
    return agg.reshape(n, heads * out_ch) + b


def kernel(x, edge_index, W1, b1, W2, b2, W3, b3, Wg1, as1, ad1, bg1, Wg2, as2, ad2, bg2):
    n = x.shape[0]
    W3p = jnp.zeros((256, 128), jnp.float32).at[:, :2].set(W3)
    b3p = jnp.zeros((128,), jnp.float32).at[:2].set(b3)
    z = _mlp(x, W1, b1, W2, b2, W3p, b3p)[:, :2]
    loops = jnp.arange(n, dtype=edge_index.dtype)
    src = jnp.concatenate([edge_index[0], loops])
    dst = jnp.concatenate([edge_index[1], loops])
    g1 = jax.nn.relu(_gat(z, src, dst, Wg1, as1, ad1, bg1, 4, 10, n))
    g2 = _gat(g1, src, dst, Wg2, as2, ad2, bg2, 1, 128, n)
    return jax.nn.sigmoid(g2) * 256.0


# speedup vs baseline: 53.0452x; 53.0452x over previous
import functools

import jax
import jax.numpy as jnp
from jax import lax
from jax.experimental import pallas as pl
from jax.experimental.pallas import tpu as pltpu
from jax.experimental.pallas import tpu_sc as plsc

N = 50000
NPAD = 51200
E = 800000
EP = 819200
W = 1024
CH = 8

_f32 = jnp.float32
_i32 = jnp.int32

_SC_PARAMS = pltpu.CompilerParams(use_tc_tiling_on_sc=False)


def _a_body(x_ref, w1, b1, w2, b2, w3p, b3p, sd1p, ts_ref, td_ref, mx_ref):
    h = jax.nn.relu(jnp.dot(x_ref[...], w1[...], preferred_element_type=_f32) + b1[...])
    h = jax.nn.relu(jnp.dot(h, w2[...], preferred_element_type=_f32) + b2[...])
    z128 = jnp.dot(h, w3p[...], preferred_element_type=_f32) + b3p[...]
    alz = jnp.dot(z128, sd1p[...], preferred_element_type=_f32)
    blk = x_ref.shape[0]
    als = alz[:, 0:4]
    ald = alz[:, 4:8]
    z2 = z128[:, 0:2]
    als_x = jnp.broadcast_to(als[:, :, None], (blk, 4, 2)).reshape(blk, 8)
    ald_x = jnp.broadcast_to(ald[:, :, None], (blk, 4, 2)).reshape(blk, 8)
    z_x = jnp.broadcast_to(z2[:, None, :], (blk, 4, 2)).reshape(blk, 8)
    zero4 = jnp.zeros((blk, 4), _f32)
    ts_ref[...] = jnp.concatenate(
        [als, als_x, zero4, jnp.ones((blk, 4), _f32), z_x, zero4], axis=1)
    td_ref[...] = jnp.concatenate([ald, ald_x, zero4], axis=1)
    mx_ref[...] = jnp.max(alz[:, 0:8], axis=0, keepdims=True)[None]


def _kernel_a(x_pad, W1, b1, W2, b2, W3p, b3p, SD1p):
    BLK = 1024
    grid = (NPAD // BLK,)
    return pl.pallas_call(
        _a_body,
        grid=grid,
        in_specs=[
            pl.BlockSpec((BLK, 128), lambda i: (i, 0)),
            pl.BlockSpec((128, 256), lambda i: (0, 0)),
            pl.BlockSpec((256,), lambda i: (0,)),
            pl.BlockSpec((256, 256), lambda i: (0, 0)),
            pl.BlockSpec((256,), lambda i: (0,)),
            pl.BlockSpec((256, 128), lambda i: (0, 0)),
            pl.BlockSpec((128,), lambda i: (0,)),
            pl.BlockSpec((128, 128), lambda i: (0, 0)),
        ],
        out_specs=[
            pl.BlockSpec((BLK, 32), lambda i: (i, 0)),
            pl.BlockSpec((BLK, 16), lambda i: (i, 0)),
            pl.BlockSpec((1, 1, 8), lambda i: (i, 0, 0)),
        ],
        out_shape=[
            jax.ShapeDtypeStruct((NPAD, 32), _f32),
            jax.ShapeDtypeStruct((NPAD, 16), _f32),
            jax.ShapeDtypeStruct((NPAD // BLK, 1, 8), _f32),
        ],
    )(x_pad, W1, b1, W2, b2, W3p, b3p, SD1p)


def _s1_body(tabs, tabd, sid_hbm, did_hbm, c1_hbm, zinit, out,
             sid_v, did_v, srows, drows, contrib, c1_v, acc, sem):
    c = lax.axis_index("c")
    s = lax.axis_index("s")
    wid = c * 16 + s
    pltpu.sync_copy(zinit.at[pl.ds(s * 1600, 1600)], acc.at[pl.ds(s * 1600, 1600)])
    pltpu.sync_copy(c1_hbm, c1_v)
    c1 = c1_v[...]
    plsc.subcore_barrier()

    def window(win, carry):
        rb = wid * 200 + win * CH
        pltpu.sync_copy(sid_hbm.at[pl.ds(rb, CH)], sid_v)
        pltpu.sync_copy(did_hbm.at[pl.ds(rb, CH)], did_v)
        descs = []
        for j in range(CH):
            descs.append(pltpu.async_copy(
                tabs.at[sid_v.at[j]], srows.at[pl.ds(j * 128, 128)], sem))
            descs.append(pltpu.async_copy(
                tabd.at[did_v.at[j]], drows.at[pl.ds(j * 128, 128)], sem))
        for dsc in descs:
            dsc.wait()

        @plsc.parallel_loop(0, W, unroll=8)
        def edge(e):
            sa = srows[e, 0:16]
            sz = srows[e, 16:32]
            sd = drows[e, :]
            t = sa + sd
            ex = jnp.exp(jnp.where(t >= 0.0, t, 0.2 * t) - c1)
            contrib[e, :] = ex * sz

        sdescs = []
        for j in range(CH):
            sdescs.append(pltpu.async_copy(
                contrib.at[pl.ds(j * 128, 128)], acc.at[did_v.at[j]], sem, add=True))
        for dsc in sdescs:
            dsc.wait()
        return carry
    lax.fori_loop(0, 25, window, 0)
    plsc.subcore_barrier()
    pltpu.sync_copy(acc.at[pl.ds(s * 1600, 1600)],
                    out.at[c].at[pl.ds(s * 1600, 1600)])


def _kernel_s1(tabs1, tabd1, sid2d, did2d, c1_arr, zinit16):
    mesh = plsc.VectorSubcoreMesh(core_axis_name="c", subcore_axis_name="s")
    f = functools.partial(
        pl.kernel,
        out_type=jax.ShapeDtypeStruct((2, NPAD, 16), _f32),
        mesh=mesh,
        compiler_params=_SC_PARAMS,
        scratch_types=[
            pltpu.VMEM((CH, 128), _i32),
            pltpu.VMEM((CH, 128), _i32),
            pltpu.VMEM((W, 32), _f32),
            pltpu.VMEM((W, 16), _f32),
            pltpu.VMEM((W, 16), _f32),
            pltpu.VMEM((16,), _f32),
            pltpu.VMEM_SHARED((NPAD, 16), _f32),
            pltpu.SemaphoreType.DMA,
        ],
    )(_s1_body)
    return f(tabs1, tabd1, sid2d, did2d, c1_arr, zinit16)


def _b_body(p1a, p1b, ts1, td1, c1r, Mr, bg1r, s2vr, d2vr,
            sta_ref, stb_ref, als2_ref, ald2_ref, mx2_ref):
    c1 = c1r[0, 0]
    blk = ts1.shape[0]
    als1 = ts1[:, 0:4]
    z = ts1[:, 20:22]
    ald1 = td1[:, 0:4]
    t = als1 + ald1
    exs = jnp.exp(jnp.where(t >= 0.0, t, 0.2 * t) - c1)
    s_h = p1a[:, 0:4] + p1b[:, 0:4] + exs
    u_self = (exs[:, :, None] * z[:, None, :]).reshape(blk, 8)
    u = p1a[:, 4:12] + p1b[:, 4:12] + u_self
    sden = jnp.broadcast_to(s_h[:, :, None], (blk, 4, 2)).reshape(blk, 8)
    agg = u / sden
    g1 = jax.nn.relu(jnp.dot(agg, Mr[...], preferred_element_type=_f32) + bg1r[...])
    row = pl.program_id(0) * blk + lax.broadcasted_iota(_i32, (blk, 1), 0)
    g1 = jnp.where(row < N, g1, 0.0)
    als2 = jnp.dot(g1, s2vr[...], preferred_element_type=_f32)
    ald2 = jnp.dot(g1, d2vr[...], preferred_element_type=_f32)
    zpad11 = jnp.zeros((blk, 11), _f32)
    one1 = jnp.ones((blk, 1), _f32)
    sta_ref[...] = jnp.concatenate([one1, g1[:, 0:20], zpad11], axis=1)
    stb_ref[...] = jnp.concatenate([one1, g1[:, 20:40], zpad11], axis=1)
    als2_ref[...] = als2
    ald2_ref[...] = ald2
    mx2_ref[...] = jnp.concatenate(
        [jnp.max(als2, axis=0, keepdims=True),
         jnp.max(ald2, axis=0, keepdims=True),
         jnp.zeros((1, 6), _f32)], axis=1)[None]


def _kernel_b(p1a, p1b, tabs1, tabd1, c1_128, M, bg1, S2v, D2v):
    BLK = 1024
    grid = (NPAD // BLK,)
    return pl.pallas_call(
        _b_body,
        grid=grid,
        in_specs=[
            pl.BlockSpec((BLK, 16), lambda i: (i, 0)),
            pl.BlockSpec((BLK, 16), lambda i: (i, 0)),
            pl.BlockSpec((BLK, 32), lambda i: (i, 0)),
            pl.BlockSpec((BLK, 16), lambda i: (i, 0)),
            pl.BlockSpec((1, 128), lambda i: (0, 0)),
            pl.BlockSpec((8, 40), lambda i: (0, 0)),
            pl.BlockSpec((40,), lambda i: (0,)),
            pl.BlockSpec((40, 1), lambda i: (0, 0)),
            pl.BlockSpec((40, 1), lambda i: (0, 0)),
        ],
        out_specs=[
            pl.BlockSpec((BLK, 32), lambda i: (i, 0)),
            pl.BlockSpec((BLK, 32), lambda i: (i, 0)),
            pl.BlockSpec((BLK, 1), lambda i: (i, 0)),
            pl.BlockSpec((BLK, 1), lambda i: (i, 0)),
            pl.BlockSpec((1, 1, 8), lambda i: (i, 0, 0)),
        ],
        out_shape=[
            jax.ShapeDtypeStruct((NPAD, 32), _f32),
            jax.ShapeDtypeStruct((NPAD, 32), _f32),
            jax.ShapeDtypeStruct((NPAD, 1), _f32),
            jax.ShapeDtypeStruct((NPAD, 1), _f32),
            jax.ShapeDtypeStruct((NPAD // BLK, 1, 8), _f32),
        ],
    )(p1a, p1b, tabs1, tabd1, c1_128, M, bg1, S2v, D2v)


W2 = 256
CH2 = W2 // 128


def _s2_body(st2cat, als2cat, ald2tab, sid_hbm, did_hbm, c2_hbm, zinit, out,
             sid_v, did_v, srows, alsv, aldv, contrib, c2_v, acc, sem):
    c = lax.axis_index("c")
    s = lax.axis_index("s")
    pltpu.sync_copy(zinit.at[pl.ds(s * 1600, 1600)], acc.at[pl.ds(s * 1600, 1600)])
    pltpu.sync_copy(c2_hbm, c2_v)
    c2 = c2_v[...]
    plsc.subcore_barrier()

    def window(win, carry):
        rb = s * 400 + win * CH2
        pltpu.sync_copy(sid_hbm.at[c].at[pl.ds(rb, CH2)], sid_v)
        pltpu.sync_copy(did_hbm.at[pl.ds(rb, CH2)], did_v)
        descs = []
        for j in range(CH2):
            descs.append(pltpu.async_copy(
                st2cat.at[sid_v.at[j]], srows.at[pl.ds(j * 128, 128)], sem))
            descs.append(pltpu.async_copy(
                als2cat.at[sid_v.at[j]], alsv.at[pl.ds(j * 128, 128)], sem))
            descs.append(pltpu.async_copy(
                ald2tab.at[did_v.at[j]], aldv.at[pl.ds(j * 128, 128)], sem))
        for dsc in descs:
            dsc.wait()

        def grp(g, carry2):
            a = alsv[pl.ds(g * 16, 16)]
            d = aldv[pl.ds(g * 16, 16)]
            t = a + d
            ex16 = jnp.exp(jnp.where(t >= 0.0, t, 0.2 * t) - c2)
            for l in range(16):
                e = g * 16 + l
                exv = jnp.broadcast_to(ex16[l], (16,))
                contrib[e, 0:16] = exv * srows[e, 0:16]
                contrib[e, 16:32] = exv * srows[e, 16:32]
            return carry2
        lax.fori_loop(0, W2 // 16, grp, 0)

        sdescs = []
        for j in range(CH2):
            sdescs.append(pltpu.async_copy(
                contrib.at[pl.ds(j * 128, 128)], acc.at[did_v.at[j]], sem, add=True))
        for dsc in sdescs:
            dsc.wait()
        return carry
    lax.fori_loop(0, EP // 16 // W2, window, 0)
    plsc.subcore_barrier()
    pltpu.sync_copy(acc.at[pl.ds(s * 1600, 1600)],
                    out.at[c].at[pl.ds(s * 1600, 1600)])


def _kernel_s2(st2cat, als2cat, ald2tab, sid3d, did2d, c2_arr, zinit32):
    mesh = plsc.VectorSubcoreMesh(core_axis_name="c", subcore_axis_name="s")
    f = functools.partial(
        pl.kernel,
        out_type=jax.ShapeDtypeStruct((2, NPAD, 32), _f32),
        mesh=mesh,
        compiler_params=_SC_PARAMS,
        scratch_types=[
            pltpu.VMEM((CH2, 128), _i32),
            pltpu.VMEM((CH2, 128), _i32),
            pltpu.VMEM((W2, 32), _f32),
            pltpu.VMEM((W2,), _f32),
            pltpu.VMEM((W2,), _f32),
            pltpu.VMEM((W2, 32), _f32),
            pltpu.VMEM((16,), _f32),
            pltpu.VMEM_SHARED((NPAD, 32), _f32),
            pltpu.SemaphoreType.DMA,
        ],
    )(_s2_body)
    return f(st2cat, als2cat, ald2tab, sid3d, did2d, c2_arr, zinit32)


def _c_body(p2a, p2b, sta, stb, als2t, ald2t, c2r, wg2, bg2r, out_ref):
    c2 = c2r[0, 0]
    als2 = als2t[...]
    ald2 = ald2t[...]
    g1 = jnp.concatenate([sta[:, 1:21], stb[:, 1:21]], axis=1)
    t = als2 + ald2
    ex = jnp.exp(jnp.where(t >= 0.0, t, 0.2 * t) - c2)
    s2 = p2a[:, 0:1] + p2b[:, 0:1] + ex
    v = jnp.concatenate([p2a[:, 1:21], p2b[:, 1:21]], axis=1) + ex * g1
    agg = v / s2
    g2 = jnp.dot(agg, wg2[...], preferred_element_type=_f32) + bg2r[...]
    out_ref[...] = jax.nn.sigmoid(g2) * 256.0


def _kernel_c(p2a, p2b, sta, stb, als2t, ald2t, c2_128, Wg2, bg2):
    BLK = 1024
    grid = (NPAD // BLK,)
    return pl.pallas_call(
        _c_body,
        grid=grid,
        in_specs=[
            pl.BlockSpec((BLK, 32), lambda i: (i, 0)),
            pl.BlockSpec((BLK, 32), lambda i: (i, 0)),
            pl.BlockSpec((BLK, 32), lambda i: (i, 0)),
            pl.BlockSpec((BLK, 32), lambda i: (i, 0)),
            pl.BlockSpec((BLK, 1), lambda i: (i, 0)),
            pl.BlockSpec((BLK, 1), lambda i: (i, 0)),
            pl.BlockSpec((1, 128), lambda i: (0, 0)),
            pl.BlockSpec((40, 128), lambda i: (0, 0)),
            pl.BlockSpec((128,), lambda i: (0,)),
        ],
        out_specs=pl.BlockSpec((BLK, 128), lambda i: (i, 0)),
        out_shape=jax.ShapeDtypeStruct((NPAD, 128), _f32),
    )(p2a, p2b, sta, stb, als2t, ald2t, c2_128, Wg2, bg2)


def kernel(x, edge_index, W1, b1, W2, b2, W3, b3, Wg1, as1, ad1, bg1, Wg2, as2, ad2, bg2):
    W3p = jnp.zeros((256, 128), _f32).at[:, :2].set(W3)
    b3p = jnp.zeros((128,), _f32).at[:2].set(b3)
    Wg1r = Wg1.reshape(2, 4, 10)
    S1m = jnp.einsum('jhc,hc->jh', Wg1r, as1)
    D1m = jnp.einsum('jhc,hc->jh', Wg1r, ad1)
    SD1p = (jnp.zeros((128, 128), _f32)
            .at[0:2, 0:4].set(S1m).at[0:2, 4:8].set(D1m))
    M4 = jnp.zeros((4, 2, 4, 10), _f32).at[jnp.arange(4), :, jnp.arange(4), :].set(
        Wg1r.transpose(1, 0, 2))
    M = M4.reshape(8, 40)
    S2v = (Wg2 @ as2[0])[:, None]
    D2v = (Wg2 @ ad2[0])[:, None]
    x_pad = jnp.zeros((NPAD, 128), _f32).at[:N].set(x)
    pad_cnt = EP - E
    src_ids = jnp.concatenate([edge_index[0], jnp.full((pad_cnt,), N, _i32)])
    dst_ids = jnp.concatenate(
        [edge_index[1], (N + jnp.arange(pad_cnt, dtype=_i32) % 1024).astype(_i32)])
    sid2d = src_ids.reshape(EP // 128, 128)
    did2d = dst_ids.reshape(EP // 128, 128)
    sid3d = jnp.stack([sid2d, sid2d + NPAD])
    zinit16 = jnp.zeros((NPAD, 16), _f32)
    zinit32 = jnp.zeros((NPAD, 32), _f32)

    tabs1, tabd1, mx1 = _kernel_a(x_pad, W1, b1, W2, b2, W3p, b3p, SD1p)
    c1 = jnp.max(mx1[:, 0, 0:4]) + jnp.max(mx1[:, 0, 4:8])
    c1 = jnp.where(c1 >= 0.0, c1, 0.2 * c1)
    c1_arr = jnp.full((16,), c1, _f32)
    c1_128 = jnp.full((1, 128), c1, _f32)

    p1 = _kernel_s1(tabs1, tabd1, sid2d, did2d, c1_arr, zinit16)

    sta, stb, als2t, ald2t, mx2 = _kernel_b(
        p1[0], p1[1], tabs1, tabd1, c1_128, M, bg1, S2v, D2v)
    c2 = jnp.max(mx2[:, 0, 0]) + jnp.max(mx2[:, 0, 1])
    c2 = jnp.where(c2 >= 0.0, c2, 0.2 * c2)
    c2_arr = jnp.full((16,), c2, _f32)
    c2_128 = jnp.full((1, 128), c2, _f32)
    st2cat = jnp.concatenate([sta, stb], axis=0)
    als2cat = jnp.concatenate([als2t[:, 0], als2t[:, 0]])
    ald2flat = ald2t[:, 0]

    p2 = _kernel_s2(st2cat, als2cat, ald2flat, sid3d, did2d, c2_arr, zinit32)

    outp = _kernel_c(p2[0], p2[1], sta, stb, als2t, ald2t, c2_128, Wg2, bg2)
    return outp[:N]


# speedup vs baseline: 70.8330x; 1.3353x over previous
import functools

import jax
import jax.numpy as jnp
from jax import lax
from jax.experimental import pallas as pl
from jax.experimental.pallas import tpu as pltpu
from jax.experimental.pallas import tpu_sc as plsc

N = 50000
NPAD = 51200
E = 800000
EP = 819200
W = 1024
CH = 8

_f32 = jnp.float32
_i32 = jnp.int32

_SC_PARAMS = pltpu.CompilerParams(use_tc_tiling_on_sc=False)


def _a_body(x_ref, w1, b1, w2, b2, w3p, b3p, sd1p, ts_ref, td_ref, mx_ref):
    h = jax.nn.relu(jnp.dot(x_ref[...], w1[...], preferred_element_type=_f32) + b1[...])
    h = jax.nn.relu(jnp.dot(h, w2[...], preferred_element_type=_f32) + b2[...])
    z128 = jnp.dot(h, w3p[...], preferred_element_type=_f32) + b3p[...]
    alz = jnp.dot(z128, sd1p[...], preferred_element_type=_f32)
    blk = x_ref.shape[0]
    als = alz[:, 0:4]
    ald = alz[:, 4:8]
    z2 = z128[:, 0:2]
    als_x = jnp.broadcast_to(als[:, :, None], (blk, 4, 2)).reshape(blk, 8)
    ald_x = jnp.broadcast_to(ald[:, :, None], (blk, 4, 2)).reshape(blk, 8)
    z_x = jnp.broadcast_to(z2[:, None, :], (blk, 4, 2)).reshape(blk, 8)
    zero4 = jnp.zeros((blk, 4), _f32)
    ts_ref[...] = jnp.concatenate(
        [als, als_x, zero4, jnp.ones((blk, 4), _f32), z_x, zero4], axis=1)
    td_ref[...] = jnp.concatenate([ald, ald_x, zero4], axis=1)
    mx_ref[...] = jnp.max(alz[:, 0:8], axis=0, keepdims=True)[None]


def _kernel_a(x_pad, W1, b1, W2, b2, W3p, b3p, SD1p):
    BLK = 1024
    grid = (NPAD // BLK,)
    return pl.pallas_call(
        _a_body,
        grid=grid,
        in_specs=[
            pl.BlockSpec((BLK, 128), lambda i: (i, 0)),
            pl.BlockSpec((128, 256), lambda i: (0, 0)),
            pl.BlockSpec((256,), lambda i: (0,)),
            pl.BlockSpec((256, 256), lambda i: (0, 0)),
            pl.BlockSpec((256,), lambda i: (0,)),
            pl.BlockSpec((256, 128), lambda i: (0, 0)),
            pl.BlockSpec((128,), lambda i: (0,)),
            pl.BlockSpec((128, 128), lambda i: (0, 0)),
        ],
        out_specs=[
            pl.BlockSpec((BLK, 32), lambda i: (i, 0)),
            pl.BlockSpec((BLK, 16), lambda i: (i, 0)),
            pl.BlockSpec((1, 1, 8), lambda i: (i, 0, 0)),
        ],
        out_shape=[
            jax.ShapeDtypeStruct((NPAD, 32), _f32),
            jax.ShapeDtypeStruct((NPAD, 16), _f32),
            jax.ShapeDtypeStruct((NPAD // BLK, 1, 8), _f32),
        ],
    )(x_pad, W1, b1, W2, b2, W3p, b3p, SD1p)


W1 = 256
NWB1 = 20
NB1 = 5


def _s1_body(tabs, tabd, sid_hbm, did_hbm, c1_hbm, zinit, out,
             sidb, didb, srows, drows, contrib, c1_v, acc, g0, g1s, ssem):
    c = lax.axis_index("c")
    s = lax.axis_index("s")
    wid = c * 16 + s
    pltpu.sync_copy(zinit.at[pl.ds(s * 1600, 1600)], acc.at[pl.ds(s * 1600, 1600)])
    pltpu.sync_copy(c1_hbm, c1_v)
    c1 = c1_v[...]
    plsc.subcore_barrier()

    def issue_g(w, b, sem):
        for k in range(W1 // 128):
            pltpu.async_copy(tabs.at[sidb.at[w * 2 + k]],
                             srows.at[pl.ds(b * W1 + k * 128, 128)], sem)
            pltpu.async_copy(tabd.at[didb.at[w * 2 + k]],
                             drows.at[pl.ds(b * W1 + k * 128, 128)], sem)

    def wait_g(b, sem):
        for k in range(W1 // 128):
            pltpu.make_async_copy(tabs.at[sidb.at[k]],
                                  srows.at[pl.ds(b * W1 + k * 128, 128)], sem).wait()
            pltpu.make_async_copy(tabd.at[didb.at[k]],
                                  drows.at[pl.ds(b * W1 + k * 128, 128)], sem).wait()

    def do_window(w, b):
        off = b * W1

        @plsc.parallel_loop(0, W1, unroll=8)
        def edge(e):
            sa = srows[off + e, 0:16]
            sz = srows[off + e, 16:32]
            sd = drows[off + e, :]
            t = sa + sd
            ex = jnp.exp(jnp.where(t >= 0.0, t, 0.2 * t) - c1)
            contrib[off + e, :] = ex * sz
        for k in range(W1 // 128):
            pltpu.async_copy(contrib.at[pl.ds(off + k * 128, 128)],
                             acc.at[didb.at[w * 2 + k]], ssem, add=True)
        for k in range(W1 // 128):
            pltpu.make_async_copy(contrib.at[pl.ds(off + k * 128, 128)],
                                  acc.at[didb.at[k]], ssem).wait()

    def blk(bk, carry):
        rb = wid * 200 + bk * (2 * NWB1)
        pltpu.sync_copy(sid_hbm.at[pl.ds(rb, 2 * NWB1)], sidb)
        pltpu.sync_copy(did_hbm.at[pl.ds(rb, 2 * NWB1)], didb)
        issue_g(0, 0, g0)

        def pair(p, carry2):
            w = 2 * p
            issue_g(w + 1, 1, g1s)
            wait_g(0, g0)
            do_window(w, 0)

            @pl.when(p < NWB1 // 2 - 1)
            def _():
                issue_g(w + 2, 0, g0)
            wait_g(1, g1s)
            do_window(w + 1, 1)
            return carry2
        lax.fori_loop(0, NWB1 // 2, pair, 0)
        return carry
    lax.fori_loop(0, NB1, blk, 0)
    plsc.subcore_barrier()
    pltpu.sync_copy(acc.at[pl.ds(s * 1600, 1600)],
                    out.at[c].at[pl.ds(s * 1600, 1600)])


def _kernel_s1(tabs1, tabd1, sid2d, did2d, c1_arr, zinit16):
    mesh = plsc.VectorSubcoreMesh(core_axis_name="c", subcore_axis_name="s")
    f = functools.partial(
        pl.kernel,
        out_type=jax.ShapeDtypeStruct((2, NPAD, 16), _f32),
        mesh=mesh,
        compiler_params=_SC_PARAMS,
        scratch_types=[
            pltpu.VMEM((2 * NWB1, 128), _i32),
            pltpu.VMEM((2 * NWB1, 128), _i32),
            pltpu.VMEM((2 * W1, 32), _f32),
            pltpu.VMEM((2 * W1, 16), _f32),
            pltpu.VMEM((2 * W1, 16), _f32),
            pltpu.VMEM((16,), _f32),
            pltpu.VMEM_SHARED((NPAD, 16), _f32),
            pltpu.SemaphoreType.DMA,
            pltpu.SemaphoreType.DMA,
            pltpu.SemaphoreType.DMA,
        ],
    )(_s1_body)
    return f(tabs1, tabd1, sid2d, did2d, c1_arr, zinit16)


def _b_body(p1a, p1b, ts1, td1, c1r, Mr, bg1r, s2vr, d2vr,
            sta_ref, stb_ref, als2_ref, ald2_ref, mx2_ref):
    c1 = c1r[0, 0]
    blk = ts1.shape[0]
    als1 = ts1[:, 0:4]
    z = ts1[:, 20:22]
    ald1 = td1[:, 0:4]
    t = als1 + ald1
    exs = jnp.exp(jnp.where(t >= 0.0, t, 0.2 * t) - c1)
    s_h = p1a[:, 0:4] + p1b[:, 0:4] + exs
    u_self = (exs[:, :, None] * z[:, None, :]).reshape(blk, 8)
    u = p1a[:, 4:12] + p1b[:, 4:12] + u_self
    sden = jnp.broadcast_to(s_h[:, :, None], (blk, 4, 2)).reshape(blk, 8)
    agg = u / sden
    g1 = jax.nn.relu(jnp.dot(agg, Mr[...], preferred_element_type=_f32) + bg1r[...])
    row = pl.program_id(0) * blk + lax.broadcasted_iota(_i32, (blk, 1), 0)
    g1 = jnp.where(row < N, g1, 0.0)
    als2 = jnp.dot(g1, s2vr[...], preferred_element_type=_f32)
    ald2 = jnp.dot(g1, d2vr[...], preferred_element_type=_f32)
    zpad11 = jnp.zeros((blk, 11), _f32)
    one1 = jnp.ones((blk, 1), _f32)
    sta_ref[...] = jnp.concatenate([one1, g1[:, 0:20], zpad11], axis=1)
    stb_ref[...] = jnp.concatenate([one1, g1[:, 20:40], zpad11], axis=1)
    als2_ref[...] = als2
    ald2_ref[...] = ald2
    mx2_ref[...] = jnp.concatenate(
        [jnp.max(als2, axis=0, keepdims=True),
         jnp.max(ald2, axis=0, keepdims=True),
         jnp.zeros((1, 6), _f32)], axis=1)[None]


def _kernel_b(p1a, p1b, tabs1, tabd1, c1_128, M, bg1, S2v, D2v):
    BLK = 1024
    grid = (NPAD // BLK,)
    return pl.pallas_call(
        _b_body,
        grid=grid,
        in_specs=[
            pl.BlockSpec((BLK, 16), lambda i: (i, 0)),
            pl.BlockSpec((BLK, 16), lambda i: (i, 0)),
            pl.BlockSpec((BLK, 32), lambda i: (i, 0)),
            pl.BlockSpec((BLK, 16), lambda i: (i, 0)),
            pl.BlockSpec((1, 128), lambda i: (0, 0)),
            pl.BlockSpec((8, 40), lambda i: (0, 0)),
            pl.BlockSpec((40,), lambda i: (0,)),
            pl.BlockSpec((40, 1), lambda i: (0, 0)),
            pl.BlockSpec((40, 1), lambda i: (0, 0)),
        ],
        out_specs=[
            pl.BlockSpec((BLK, 32), lambda i: (i, 0)),
            pl.BlockSpec((BLK, 32), lambda i: (i, 0)),
            pl.BlockSpec((BLK, 1), lambda i: (i, 0)),
            pl.BlockSpec((BLK, 1), lambda i: (i, 0)),
            pl.BlockSpec((1, 1, 8), lambda i: (i, 0, 0)),
        ],
        out_shape=[
            jax.ShapeDtypeStruct((NPAD, 32), _f32),
            jax.ShapeDtypeStruct((NPAD, 32), _f32),
            jax.ShapeDtypeStruct((NPAD, 1), _f32),
            jax.ShapeDtypeStruct((NPAD, 1), _f32),
            jax.ShapeDtypeStruct((NPAD // BLK, 1, 8), _f32),
        ],
    )(p1a, p1b, tabs1, tabd1, c1_128, M, bg1, S2v, D2v)


W2 = 128
NB2 = 20
NWB2 = 20


def _s2_body(sta, stb, als2tab, ald2tab, sid_hbm, did_hbm, c2_hbm, zinit, out,
             sidb, didb, srows, alsv, aldv, contrib, c2_v, acc, g0, g1s, ssem):
    c = lax.axis_index("c")
    s = lax.axis_index("s")
    pltpu.sync_copy(zinit.at[pl.ds(s * 1600, 1600)], acc.at[pl.ds(s * 1600, 1600)])
    pltpu.sync_copy(c2_hbm, c2_v)
    c2 = c2_v[...]
    plsc.subcore_barrier()

    def issue_g(w, b, sem):
        @pl.when(c == 0)
        def _():
            pltpu.async_copy(sta.at[sidb.at[w]],
                             srows.at[pl.ds(b * W2, 128)], sem)

        @pl.when(c == 1)
        def _():
            pltpu.async_copy(stb.at[sidb.at[w]],
                             srows.at[pl.ds(b * W2, 128)], sem)
        pltpu.async_copy(als2tab.at[sidb.at[w]], alsv.at[pl.ds(b * W2, 128)], sem)
        pltpu.async_copy(ald2tab.at[didb.at[w]], aldv.at[pl.ds(b * W2, 128)], sem)

    def wait_g(b, sem):
        pltpu.make_async_copy(sta.at[sidb.at[0]],
                              srows.at[pl.ds(b * W2, 128)], sem).wait()
        pltpu.make_async_copy(als2tab.at[sidb.at[0]],
                              alsv.at[pl.ds(b * W2, 128)], sem).wait()
        pltpu.make_async_copy(ald2tab.at[didb.at[0]],
                              aldv.at[pl.ds(b * W2, 128)], sem).wait()

    def do_window(w, b):
        off = b * W2

        def grp(g, carry2):
            a = alsv[pl.ds(off + g * 16, 16)]
            d = aldv[pl.ds(off + g * 16, 16)]
            t = a + d
            ex16 = jnp.exp(jnp.where(t >= 0.0, t, 0.2 * t) - c2)
            for l in range(16):
                e = off + g * 16 + l
                exv = jnp.broadcast_to(ex16[l], (16,))
                contrib[e, 0:16] = exv * srows[e, 0:16]
                contrib[e, 16:32] = exv * srows[e, 16:32]
            return carry2
        lax.fori_loop(0, W2 // 16, grp, 0)
        pltpu.async_copy(contrib.at[pl.ds(off, 128)],
                         acc.at[didb.at[w]], ssem, add=True)
        pltpu.make_async_copy(contrib.at[pl.ds(off, 128)],
                              acc.at[didb.at[0]], ssem).wait()

    def blk(bk, carry):
        rb = s * 400 + bk * NWB2
        pltpu.sync_copy(sid_hbm.at[pl.ds(rb, NWB2)], sidb)
        pltpu.sync_copy(did_hbm.at[pl.ds(rb, NWB2)], didb)
        issue_g(0, 0, g0)

        def pair(p, carry2):
            w = 2 * p
            issue_g(w + 1, 1, g1s)
            wait_g(0, g0)
            do_window(w, 0)

            @pl.when(p < NWB2 // 2 - 1)
            def _():
                issue_g(w + 2, 0, g0)
            wait_g(1, g1s)
            do_window(w + 1, 1)
            return carry2
        lax.fori_loop(0, NWB2 // 2, pair, 0)
        return carry
    lax.fori_loop(0, NB2, blk, 0)
    plsc.subcore_barrier()
    pltpu.sync_copy(acc.at[pl.ds(s * 1600, 1600)],
                    out.at[c].at[pl.ds(s * 1600, 1600)])


def _kernel_s2(sta, stb, als2tab, ald2tab, sid2d, did2d, c2_arr, zinit32):
    mesh = plsc.VectorSubcoreMesh(core_axis_name="c", subcore_axis_name="s")
    f = functools.partial(
        pl.kernel,
        out_type=jax.ShapeDtypeStruct((2, NPAD, 32), _f32),
        mesh=mesh,
        compiler_params=_SC_PARAMS,
        scratch_types=[
            pltpu.VMEM((NWB2, 128), _i32),
            pltpu.VMEM((NWB2, 128), _i32),
            pltpu.VMEM((2 * W2, 32), _f32),
            pltpu.VMEM((2 * W2,), _f32),
            pltpu.VMEM((2 * W2,), _f32),
            pltpu.VMEM((2 * W2, 32), _f32),
            pltpu.VMEM((16,), _f32),
            pltpu.VMEM_SHARED((NPAD, 32), _f32),
            pltpu.SemaphoreType.DMA,
            pltpu.SemaphoreType.DMA,
            pltpu.SemaphoreType.DMA,
        ],
    )(_s2_body)
    return f(sta, stb, als2tab, ald2tab, sid2d, did2d, c2_arr, zinit32)


def _c_body(p2a, p2b, sta, stb, als2t, ald2t, c2r, wg2, bg2r, out_ref):
    c2 = c2r[0, 0]
    als2 = als2t[...]
    ald2 = ald2t[...]
    g1 = jnp.concatenate([sta[:, 1:21], stb[:, 1:21]], axis=1)
    t = als2 + ald2
    ex = jnp.exp(jnp.where(t >= 0.0, t, 0.2 * t) - c2)
    s2 = p2a[:, 0:1] + p2b[:, 0:1] + ex
    v = jnp.concatenate([p2a[:, 1:21], p2b[:, 1:21]], axis=1) + ex * g1
    agg = v / s2
    g2 = jnp.dot(agg, wg2[...], preferred_element_type=_f32) + bg2r[...]
    out_ref[...] = jax.nn.sigmoid(g2) * 256.0


def _kernel_c(p2a, p2b, sta, stb, als2t, ald2t, c2_128, Wg2, bg2):
    BLK = 1024
    grid = (NPAD // BLK,)
    return pl.pallas_call(
        _c_body,
        grid=grid,
        in_specs=[
            pl.BlockSpec((BLK, 32), lambda i: (i, 0)),
            pl.BlockSpec((BLK, 32), lambda i: (i, 0)),
            pl.BlockSpec((BLK, 32), lambda i: (i, 0)),
            pl.BlockSpec((BLK, 32), lambda i: (i, 0)),
            pl.BlockSpec((BLK, 1), lambda i: (i, 0)),
            pl.BlockSpec((BLK, 1), lambda i: (i, 0)),
            pl.BlockSpec((1, 128), lambda i: (0, 0)),
            pl.BlockSpec((40, 128), lambda i: (0, 0)),
            pl.BlockSpec((128,), lambda i: (0,)),
        ],
        out_specs=pl.BlockSpec((BLK, 128), lambda i: (i, 0)),
        out_shape=jax.ShapeDtypeStruct((NPAD, 128), _f32),
    )(p2a, p2b, sta, stb, als2t, ald2t, c2_128, Wg2, bg2)


def kernel(x, edge_index, W1, b1, W2, b2, W3, b3, Wg1, as1, ad1, bg1, Wg2, as2, ad2, bg2):
    W3p = jnp.zeros((256, 128), _f32).at[:, :2].set(W3)
    b3p = jnp.zeros((128,), _f32).at[:2].set(b3)
    Wg1r = Wg1.reshape(2, 4, 10)
    S1m = jnp.einsum('jhc,hc->jh', Wg1r, as1)
    D1m = jnp.einsum('jhc,hc->jh', Wg1r, ad1)
    SD1p = (jnp.zeros((128, 128), _f32)
            .at[0:2, 0:4].set(S1m).at[0:2, 4:8].set(D1m))
    M4 = jnp.zeros((4, 2, 4, 10), _f32).at[jnp.arange(4), :, jnp.arange(4), :].set(
        Wg1r.transpose(1, 0, 2))
    M = M4.reshape(8, 40)
    S2v = (Wg2 @ as2[0])[:, None]
    D2v = (Wg2 @ ad2[0])[:, None]
    x_pad = jnp.zeros((NPAD, 128), _f32).at[:N].set(x)
    pad_cnt = EP - E
    src_ids = jnp.concatenate([edge_index[0], jnp.full((pad_cnt,), N, _i32)])
    dst_ids = jnp.concatenate(
        [edge_index[1], (N + jnp.arange(pad_cnt, dtype=_i32) % 1024).astype(_i32)])
    sid2d = src_ids.reshape(EP // 128, 128)
    did2d = dst_ids.reshape(EP // 128, 128)
    zinit16 = jnp.zeros((NPAD, 16), _f32)
    zinit32 = jnp.zeros((NPAD, 32), _f32)

    tabs1, tabd1, mx1 = _kernel_a(x_pad, W1, b1, W2, b2, W3p, b3p, SD1p)
    c1 = jnp.max(mx1[:, 0, 0:4]) + jnp.max(mx1[:, 0, 4:8])
    c1 = jnp.where(c1 >= 0.0, c1, 0.2 * c1)
    c1_arr = jnp.full((16,), c1, _f32)
    c1_128 = jnp.full((1, 128), c1, _f32)

    p1 = _kernel_s1(tabs1, tabd1, sid2d, did2d, c1_arr, zinit16)

    sta, stb, als2t, ald2t, mx2 = _kernel_b(
        p1[0], p1[1], tabs1, tabd1, c1_128, M, bg1, S2v, D2v)
    c2 = jnp.max(mx2[:, 0, 0]) + jnp.max(mx2[:, 0, 1])
    c2 = jnp.where(c2 >= 0.0, c2, 0.2 * c2)
    c2_arr = jnp.full((16,), c2, _f32)
    c2_128 = jnp.full((1, 128), c2, _f32)
    als2flat = als2t[:, 0]
    ald2flat = ald2t[:, 0]

    p2 = _kernel_s2(sta, stb, als2flat, ald2flat, sid2d, did2d, c2_arr, zinit32)

    outp = _kernel_c(p2[0], p2[1], sta, stb, als2t, ald2t, c2_128, Wg2, bg2)
    return outp[:N]


# speedup vs baseline: 87.0294x; 1.2287x over previous
import functools

import jax
import jax.numpy as jnp
from jax import lax
from jax.experimental import pallas as pl
from jax.experimental.pallas import tpu as pltpu
from jax.experimental.pallas import tpu_sc as plsc

N = 50000
NPAD = 51200
E = 800000
EP = 819200
W = 1024
CH = 8

_f32 = jnp.float32
_i32 = jnp.int32

_SC_PARAMS = pltpu.CompilerParams(use_tc_tiling_on_sc=False)


def _a_body(x_ref, w1, b1, w2, b2, w3p, b3p, p1m, q1m, pdm, ts_ref, td_ref, mx_ref):
    h = jax.nn.relu(jnp.dot(x_ref[...], w1[...], preferred_element_type=_f32) + b1[...])
    h = jax.nn.relu(jnp.dot(h, w2[...], preferred_element_type=_f32) + b2[...])
    z128 = jnp.dot(h, w3p[...], preferred_element_type=_f32) + b3p[...]
    ts = jnp.dot(z128, p1m[...], preferred_element_type=_f32) + q1m[...]
    td = jnp.dot(z128, pdm[...], preferred_element_type=_f32)
    ts_ref[...] = ts
    td_ref[...] = td
    mx_ref[...] = jnp.concatenate(
        [jnp.max(ts[:, 0:4], axis=0, keepdims=True),
         jnp.max(td[:, 0:4], axis=0, keepdims=True)], axis=1)[None]


def _kernel_a(x_pad, W1, b1, W2, b2, W3p, b3p, P1m, Q1m, PDm):
    BLK = 1024
    grid = (NPAD // BLK,)
    return pl.pallas_call(
        _a_body,
        grid=grid,
        in_specs=[
            pl.BlockSpec((BLK, 128), lambda i: (i, 0)),
            pl.BlockSpec((128, 256), lambda i: (0, 0)),
            pl.BlockSpec((256,), lambda i: (0,)),
            pl.BlockSpec((256, 256), lambda i: (0, 0)),
            pl.BlockSpec((256,), lambda i: (0,)),
            pl.BlockSpec((256, 128), lambda i: (0, 0)),
            pl.BlockSpec((128,), lambda i: (0,)),
            pl.BlockSpec((128, 32), lambda i: (0, 0)),
            pl.BlockSpec((32,), lambda i: (0,)),
            pl.BlockSpec((128, 16), lambda i: (0, 0)),
        ],
        out_specs=[
            pl.BlockSpec((BLK, 32), lambda i: (i, 0)),
            pl.BlockSpec((BLK, 16), lambda i: (i, 0)),
            pl.BlockSpec((1, 1, 8), lambda i: (i, 0, 0)),
        ],
        out_shape=[
            jax.ShapeDtypeStruct((NPAD, 32), _f32),
            jax.ShapeDtypeStruct((NPAD, 16), _f32),
            jax.ShapeDtypeStruct((NPAD // BLK, 1, 8), _f32),
        ],
    )(x_pad, W1, b1, W2, b2, W3p, b3p, P1m, Q1m, PDm)


W1 = 256
NWB1 = 20
NB1 = 5


def _s1_body(tabs, tabd, sid_hbm, did_hbm, c1_hbm, zinit, out,
             sidb, didb, srows, drows, contrib, c1_v, acc, g0, g1s, s0, s1sem):
    c = lax.axis_index("c")
    s = lax.axis_index("s")
    wid = c * 16 + s
    pltpu.sync_copy(zinit.at[pl.ds(s * 1600, 1600)], acc.at[pl.ds(s * 1600, 1600)])
    pltpu.sync_copy(c1_hbm, c1_v)
    c1 = c1_v[...]
    plsc.subcore_barrier()

    def issue_g(w, b, sem):
        for k in range(W1 // 128):
            pltpu.async_copy(tabs.at[sidb.at[w * 2 + k]],
                             srows.at[pl.ds(b * W1 + k * 128, 128)], sem)
            pltpu.async_copy(tabd.at[didb.at[w * 2 + k]],
                             drows.at[pl.ds(b * W1 + k * 128, 128)], sem)

    def wait_g(b, sem):
        for k in range(W1 // 128):
            pltpu.make_async_copy(tabs.at[sidb.at[k]],
                                  srows.at[pl.ds(b * W1 + k * 128, 128)], sem).wait()
            pltpu.make_async_copy(tabd.at[didb.at[k]],
                                  drows.at[pl.ds(b * W1 + k * 128, 128)], sem).wait()

    def do_window(w, b, sem, not_first):
        off = b * W1

        @pl.when(not_first)
        def _():
            for k in range(W1 // 128):
                pltpu.make_async_copy(contrib.at[pl.ds(off + k * 128, 128)],
                                      acc.at[didb.at[k]], sem).wait()

        @plsc.parallel_loop(0, W1, unroll=8)
        def edge(e):
            sa = srows[off + e, 0:16]
            sz = srows[off + e, 16:32]
            sd = drows[off + e, :]
            t = sa + sd
            ex = jnp.exp(jnp.where(t >= 0.0, t, 0.2 * t) - c1)
            contrib[off + e, :] = ex * sz
        for k in range(W1 // 128):
            pltpu.async_copy(contrib.at[pl.ds(off + k * 128, 128)],
                             acc.at[didb.at[w * 2 + k]], sem, add=True)

    def blk(bk, carry):
        rb = wid * 200 + bk * (2 * NWB1)
        pltpu.sync_copy(sid_hbm.at[pl.ds(rb, 2 * NWB1)], sidb)
        pltpu.sync_copy(did_hbm.at[pl.ds(rb, 2 * NWB1)], didb)
        issue_g(0, 0, g0)

        def pair(p, carry2):
            w = 2 * p
            nf = jnp.logical_or(bk > 0, p > 0)
            issue_g(w + 1, 1, g1s)
            wait_g(0, g0)
            do_window(w, 0, s0, nf)

            @pl.when(p < NWB1 // 2 - 1)
            def _():
                issue_g(w + 2, 0, g0)
            wait_g(1, g1s)
            do_window(w + 1, 1, s1sem, nf)
            return carry2
        lax.fori_loop(0, NWB1 // 2, pair, 0)
        return carry
    lax.fori_loop(0, NB1, blk, 0)
    for k in range(W1 // 128):
        pltpu.make_async_copy(contrib.at[pl.ds(k * 128, 128)],
                              acc.at[didb.at[k]], s0).wait()
    for k in range(W1 // 128):
        pltpu.make_async_copy(contrib.at[pl.ds(W1 + k * 128, 128)],
                              acc.at[didb.at[k]], s1sem).wait()
    plsc.subcore_barrier()
    pltpu.sync_copy(acc.at[pl.ds(s * 1600, 1600)],
                    out.at[c].at[pl.ds(s * 1600, 1600)])


def _kernel_s1(tabs1, tabd1, sid2d, did2d, c1_arr, zinit16):
    mesh = plsc.VectorSubcoreMesh(core_axis_name="c", subcore_axis_name="s")
    f = functools.partial(
        pl.kernel,
        out_type=jax.ShapeDtypeStruct((2, NPAD, 16), _f32),
        mesh=mesh,
        compiler_params=_SC_PARAMS,
        scratch_types=[
            pltpu.VMEM((2 * NWB1, 128), _i32),
            pltpu.VMEM((2 * NWB1, 128), _i32),
            pltpu.VMEM((2 * W1, 32), _f32),
            pltpu.VMEM((2 * W1, 16), _f32),
            pltpu.VMEM((2 * W1, 16), _f32),
            pltpu.VMEM((16,), _f32),
            pltpu.VMEM_SHARED((NPAD, 16), _f32),
            pltpu.SemaphoreType.DMA,
            pltpu.SemaphoreType.DMA,
            pltpu.SemaphoreType.DMA,
            pltpu.SemaphoreType.DMA,
        ],
    )(_s1_body)
    return f(tabs1, tabd1, sid2d, did2d, c1_arr, zinit16)


def _b_body(p1a, p1b, ts1, td1, c1r, Mr, bg1r, s2vr, d2vr, r42r, kar, kbr, qar,
            sta_ref, stb_ref, als2_ref, ald2_ref, mx2_ref):
    c1 = c1r[0, 0]
    t = ts1[:, 0:4] + td1[:, 0:4]
    exs = jnp.exp(jnp.where(t >= 0.0, t, 0.2 * t) - c1)
    s_h = p1a[:, 0:4] + p1b[:, 0:4] + exs
    exs_x = jnp.dot(exs, r42r[...], preferred_element_type=_f32)
    u_self = exs_x * ts1[:, 20:28]
    u = p1a[:, 4:12] + p1b[:, 4:12] + u_self
    sden = jnp.dot(s_h, r42r[...], preferred_element_type=_f32)
    agg = u / sden
    g1 = jax.nn.relu(jnp.dot(agg, Mr[...], preferred_element_type=_f32) + bg1r[...])
    blk = ts1.shape[0]
    row = pl.program_id(0) * blk + lax.broadcasted_iota(_i32, (blk, 1), 0)
    g1 = jnp.where(row < N, g1, 0.0)
    als2 = jnp.dot(g1, s2vr[...], preferred_element_type=_f32)
    ald2 = jnp.dot(g1, d2vr[...], preferred_element_type=_f32)
    sta_ref[...] = jnp.dot(g1, kar[...], preferred_element_type=_f32) + qar[...]
    stb_ref[...] = jnp.dot(g1, kbr[...], preferred_element_type=_f32) + qar[...]
    als2_ref[...] = als2
    ald2_ref[...] = ald2
    mx2_ref[...] = jnp.concatenate(
        [jnp.max(als2, axis=0, keepdims=True),
         jnp.max(ald2, axis=0, keepdims=True),
         jnp.zeros((1, 6), _f32)], axis=1)[None]


def _kernel_b(p1a, p1b, tabs1, tabd1, c1_128, M, bg1, S2v, D2v, R42, Ka, Kb, Qa):
    BLK = 1024
    grid = (NPAD // BLK,)
    return pl.pallas_call(
        _b_body,
        grid=grid,
        in_specs=[
            pl.BlockSpec((BLK, 16), lambda i: (i, 0)),
            pl.BlockSpec((BLK, 16), lambda i: (i, 0)),
            pl.BlockSpec((BLK, 32), lambda i: (i, 0)),
            pl.BlockSpec((BLK, 16), lambda i: (i, 0)),
            pl.BlockSpec((1, 128), lambda i: (0, 0)),
            pl.BlockSpec((8, 40), lambda i: (0, 0)),
            pl.BlockSpec((40,), lambda i: (0,)),
            pl.BlockSpec((40, 1), lambda i: (0, 0)),
            pl.BlockSpec((40, 1), lambda i: (0, 0)),
            pl.BlockSpec((4, 8), lambda i: (0, 0)),
            pl.BlockSpec((40, 32), lambda i: (0, 0)),
            pl.BlockSpec((40, 32), lambda i: (0, 0)),
            pl.BlockSpec((32,), lambda i: (0,)),
        ],
        out_specs=[
            pl.BlockSpec((BLK, 32), lambda i: (i, 0)),
            pl.BlockSpec((BLK, 32), lambda i: (i, 0)),
            pl.BlockSpec((BLK, 1), lambda i: (i, 0)),
            pl.BlockSpec((BLK, 1), lambda i: (i, 0)),
            pl.BlockSpec((1, 1, 8), lambda i: (i, 0, 0)),
        ],
        out_shape=[
            jax.ShapeDtypeStruct((NPAD, 32), _f32),
            jax.ShapeDtypeStruct((NPAD, 32), _f32),
            jax.ShapeDtypeStruct((NPAD, 1), _f32),
            jax.ShapeDtypeStruct((NPAD, 1), _f32),
            jax.ShapeDtypeStruct((NPAD // BLK, 1, 8), _f32),
        ],
    )(p1a, p1b, tabs1, tabd1, c1_128, M, bg1, S2v, D2v, R42, Ka, Kb, Qa)


W2 = 128
NB2 = 20
NWB2 = 20


def _s2_body(sta, stb, als2tab, ald2tab, sid_hbm, did_hbm, c2_hbm, zinit, out,
             sidb, didb, srows, alsv, aldv, contrib, c2_v, acc, g0, g1s, s0, s1sem):
    c = lax.axis_index("c")
    s = lax.axis_index("s")
    pltpu.sync_copy(zinit.at[pl.ds(s * 1600, 1600)], acc.at[pl.ds(s * 1600, 1600)])
    pltpu.sync_copy(c2_hbm, c2_v)
    c2 = c2_v[...]
    plsc.subcore_barrier()

    def issue_g(w, b, sem):
        @pl.when(c == 0)
        def _():
            pltpu.async_copy(sta.at[sidb.at[w]],
                             srows.at[pl.ds(b * W2, 128)], sem)

        @pl.when(c == 1)
        def _():
            pltpu.async_copy(stb.at[sidb.at[w]],
                             srows.at[pl.ds(b * W2, 128)], sem)
        pltpu.async_copy(als2tab.at[sidb.at[w]], alsv.at[pl.ds(b * W2, 128)], sem)
        pltpu.async_copy(ald2tab.at[didb.at[w]], aldv.at[pl.ds(b * W2, 128)], sem)

    def wait_g(b, sem):
        pltpu.make_async_copy(sta.at[sidb.at[0]],
                              srows.at[pl.ds(b * W2, 128)], sem).wait()
        pltpu.make_async_copy(als2tab.at[sidb.at[0]],
                              alsv.at[pl.ds(b * W2, 128)], sem).wait()
        pltpu.make_async_copy(ald2tab.at[didb.at[0]],
                              aldv.at[pl.ds(b * W2, 128)], sem).wait()

    def do_window(w, b, sem, not_first):
        off = b * W2

        @pl.when(not_first)
        def _():
            pltpu.make_async_copy(contrib.at[pl.ds(off, 128)],
                                  acc.at[didb.at[0]], sem).wait()

        def grp(g, carry2):
            a = alsv[pl.ds(off + g * 16, 16)]
            d = aldv[pl.ds(off + g * 16, 16)]
            t = a + d
            ex16 = jnp.exp(jnp.where(t >= 0.0, t, 0.2 * t) - c2)
            for l in range(16):
                e = off + g * 16 + l
                exv = jnp.broadcast_to(ex16[l], (16,))
                contrib[e, 0:16] = exv * srows[e, 0:16]
                contrib[e, 16:32] = exv * srows[e, 16:32]
            return carry2
        lax.fori_loop(0, W2 // 16, grp, 0)
        pltpu.async_copy(contrib.at[pl.ds(off, 128)],
                         acc.at[didb.at[w]], sem, add=True)

    def blk(bk, carry):
        rb = s * 400 + bk * NWB2
        pltpu.sync_copy(sid_hbm.at[pl.ds(rb, NWB2)], sidb)
        pltpu.sync_copy(did_hbm.at[pl.ds(rb, NWB2)], didb)
        issue_g(0, 0, g0)

        def pair(p, carry2):
            w = 2 * p
            nf = jnp.logical_or(bk > 0, p > 0)
            issue_g(w + 1, 1, g1s)
            wait_g(0, g0)
            do_window(w, 0, s0, nf)

            @pl.when(p < NWB2 // 2 - 1)
            def _():
                issue_g(w + 2, 0, g0)
            wait_g(1, g1s)
            do_window(w + 1, 1, s1sem, nf)
            return carry2
        lax.fori_loop(0, NWB2 // 2, pair, 0)
        return carry
    lax.fori_loop(0, NB2, blk, 0)
    pltpu.make_async_copy(contrib.at[pl.ds(0, 128)],
                          acc.at[didb.at[0]], s0).wait()
    pltpu.make_async_copy(contrib.at[pl.ds(W2, 128)],
                          acc.at[didb.at[0]], s1sem).wait()
    plsc.subcore_barrier()
    pltpu.sync_copy(acc.at[pl.ds(s * 1600, 1600)],
                    out.at[c].at[pl.ds(s * 1600, 1600)])


def _kernel_s2(sta, stb, als2tab, ald2tab, sid2d, did2d, c2_arr, zinit32):
    mesh = plsc.VectorSubcoreMesh(core_axis_name="c", subcore_axis_name="s")
    f = functools.partial(
        pl.kernel,
        out_type=jax.ShapeDtypeStruct((2, NPAD, 32), _f32),
        mesh=mesh,
        compiler_params=_SC_PARAMS,
        scratch_types=[
            pltpu.VMEM((NWB2, 128), _i32),
            pltpu.VMEM((NWB2, 128), _i32),
            pltpu.VMEM((2 * W2, 32), _f32),
            pltpu.VMEM((2 * W2,), _f32),
            pltpu.VMEM((2 * W2,), _f32),
            pltpu.VMEM((2 * W2, 32), _f32),
            pltpu.VMEM((16,), _f32),
            pltpu.VMEM_SHARED((NPAD, 32), _f32),
            pltpu.SemaphoreType.DMA,
            pltpu.SemaphoreType.DMA,
            pltpu.SemaphoreType.DMA,
            pltpu.SemaphoreType.DMA,
        ],
    )(_s2_body)
    return f(sta, stb, als2tab, ald2tab, sid2d, did2d, c2_arr, zinit32)


def _c_body(p2a, p2b, sta, stb, als2t, ald2t, c2r, wg2, bg2r, out_ref):
    c2 = c2r[0, 0]
    als2 = als2t[...]
    ald2 = ald2t[...]
    g1 = jnp.concatenate([sta[:, 1:21], stb[:, 1:21]], axis=1)
    t = als2 + ald2
    ex = jnp.exp(jnp.where(t >= 0.0, t, 0.2 * t) - c2)
    s2 = p2a[:, 0:1] + p2b[:, 0:1] + ex
    v = jnp.concatenate([p2a[:, 1:21], p2b[:, 1:21]], axis=1) + ex * g1
    agg = v / s2
    g2 = jnp.dot(agg, wg2[...], preferred_element_type=_f32) + bg2r[...]
    out_ref[...] = jax.nn.sigmoid(g2) * 256.0


def _kernel_c(p2a, p2b, sta, stb, als2t, ald2t, c2_128, Wg2, bg2):
    BLK = 1024
    grid = (NPAD // BLK,)
    return pl.pallas_call(
        _c_body,
        grid=grid,
        in_specs=[
            pl.BlockSpec((BLK, 32), lambda i: (i, 0)),
            pl.BlockSpec((BLK, 32), lambda i: (i, 0)),
            pl.BlockSpec((BLK, 32), lambda i: (i, 0)),
            pl.BlockSpec((BLK, 32), lambda i: (i, 0)),
            pl.BlockSpec((BLK, 1), lambda i: (i, 0)),
            pl.BlockSpec((BLK, 1), lambda i: (i, 0)),
            pl.BlockSpec((1, 128), lambda i: (0, 0)),
            pl.BlockSpec((40, 128), lambda i: (0, 0)),
            pl.BlockSpec((128,), lambda i: (0,)),
        ],
        out_specs=pl.BlockSpec((BLK, 128), lambda i: (i, 0)),
        out_shape=jax.ShapeDtypeStruct((NPAD, 128), _f32),
    )(p2a, p2b, sta, stb, als2t, ald2t, c2_128, Wg2, bg2)


def kernel(x, edge_index, W1, b1, W2, b2, W3, b3, Wg1, as1, ad1, bg1, Wg2, as2, ad2, bg2):
    W3p = jnp.zeros((256, 128), _f32).at[:, :2].set(W3)
    b3p = jnp.zeros((128,), _f32).at[:2].set(b3)
    Wg1r = Wg1.reshape(2, 4, 10)
    S1m = jnp.einsum('jhc,hc->jh', Wg1r, as1)
    D1m = jnp.einsum('jhc,hc->jh', Wg1r, ad1)
    hh = jnp.arange(4)
    R42 = jnp.zeros((4, 8), _f32).at[hh, 2 * hh].set(1.0).at[hh, 2 * hh + 1].set(1.0)
    kk = jnp.arange(8)
    R22 = jnp.zeros((2, 8), _f32).at[kk % 2, kk].set(1.0)
    P1m = (jnp.zeros((128, 32), _f32)
           .at[0:2, 0:4].set(S1m)
           .at[0:2, 4:12].set(S1m @ R42)
           .at[0:2, 20:28].set(R22))
    Q1m = jnp.zeros((32,), _f32).at[16:20].set(1.0)
    PDm = (jnp.zeros((128, 16), _f32)
           .at[0:2, 0:4].set(D1m)
           .at[0:2, 4:12].set(D1m @ R42))
    i20 = jnp.arange(20)
    Ka = jnp.zeros((40, 32), _f32).at[i20, 1 + i20].set(1.0)
    Kb = jnp.zeros((40, 32), _f32).at[20 + i20, 1 + i20].set(1.0)
    Qa = jnp.zeros((32,), _f32).at[0].set(1.0)
    M4 = jnp.zeros((4, 2, 4, 10), _f32).at[jnp.arange(4), :, jnp.arange(4), :].set(
        Wg1r.transpose(1, 0, 2))
    M = M4.reshape(8, 40)
    S2v = (Wg2 @ as2[0])[:, None]
    D2v = (Wg2 @ ad2[0])[:, None]
    x_pad = jnp.zeros((NPAD, 128), _f32).at[:N].set(x)
    pad_cnt = EP - E
    src_ids = jnp.concatenate([edge_index[0], jnp.full((pad_cnt,), N, _i32)])
    dst_ids = jnp.concatenate(
        [edge_index[1], (N + jnp.arange(pad_cnt, dtype=_i32) % 1024).astype(_i32)])
    sid2d = src_ids.reshape(EP // 128, 128)
    did2d = dst_ids.reshape(EP // 128, 128)
    zinit16 = jnp.zeros((NPAD, 16), _f32)
    zinit32 = jnp.zeros((NPAD, 32), _f32)

    tabs1, tabd1, mx1 = _kernel_a(x_pad, W1, b1, W2, b2, W3p, b3p, P1m, Q1m, PDm)
    c1 = jnp.max(mx1[:, 0, 0:4]) + jnp.max(mx1[:, 0, 4:8])
    c1 = jnp.where(c1 >= 0.0, c1, 0.2 * c1)
    c1_arr = jnp.full((16,), c1, _f32)
    c1_128 = jnp.full((1, 128), c1, _f32)

    p1 = _kernel_s1(tabs1, tabd1, sid2d, did2d, c1_arr, zinit16)

    sta, stb, als2t, ald2t, mx2 = _kernel_b(
        p1[0], p1[1], tabs1, tabd1, c1_128, M, bg1, S2v, D2v, R42, Ka, Kb, Qa)
    c2 = jnp.max(mx2[:, 0, 0]) + jnp.max(mx2[:, 0, 1])
    c2 = jnp.where(c2 >= 0.0, c2, 0.2 * c2)
    c2_arr = jnp.full((16,), c2, _f32)
    c2_128 = jnp.full((1, 128), c2, _f32)
    als2flat = als2t[:, 0]
    ald2flat = ald2t[:, 0]

    p2 = _kernel_s2(sta, stb, als2flat, ald2flat, sid2d, did2d, c2_arr, zinit32)

    outp = _kernel_c(p2[0], p2[1], sta, stb, als2t, ald2t, c2_128, Wg2, bg2)
    return outp[:N]
